# baseline (speedup 1.0000x reference)
import functools

import jax
import jax.numpy as jnp
from jax import lax
from jax.experimental import pallas as pl
from jax.experimental.pallas import tpu as pltpu
from jax.experimental.pallas import tpu_sc as plsc

HIST = 200
BATCH = 4096
H = 128
NC = 2
NS = 16
CH = 128
NCHUNK = BATCH // CH
LANES = 16
NV = H // LANES
GSZ = 4
PPG = 25
FULL = 6
QCH = NCHUNK // GSZ
TCH = FULL * NCHUNK + QCH
NBUF = 5
NTRIP = TCH // NBUF

_mesh = plsc.VectorSubcoreMesh(
    core_axis_name="c", subcore_axis_name="s", num_cores=NC, num_subcores=NS
)


@functools.partial(
    pl.kernel,
    mesh=_mesh,
    out_type=jax.ShapeDtypeStruct((HIST, H), jnp.float32),
    scratch_types=[
        pltpu.VMEM((TCH, CH), jnp.int32),
        pltpu.VMEM((CH, H), jnp.float32),
        pltpu.VMEM((CH, H), jnp.float32),
        pltpu.VMEM((CH, H), jnp.float32),
        pltpu.VMEM((CH, H), jnp.float32),
        pltpu.VMEM((CH, H), jnp.float32),
        pltpu.VMEM((FULL + 1, H), jnp.float32),
        pltpu.VMEM((GSZ, H), jnp.float32),
        pltpu.VMEM_SHARED((GSZ, GSZ, H), jnp.float32),
        pltpu.SemaphoreType.DMA,
        pltpu.SemaphoreType.DMA,
        pltpu.SemaphoreType.DMA,
        pltpu.SemaphoreType.DMA,
        pltpu.SemaphoreType.DMA,
        pltpu.SemaphoreType.DMA,
    ],
)
def _embed_sum(idx_hbm, table_hbm, out_hbm, idx_v, buf_a, buf_b, buf_c,
               buf_d, buf_e, acc_v, red_v, part_sh, sem_a, sem_b, sem_c,
               sem_d, sem_e, sem_o):
    c = lax.axis_index("c")
    s = lax.axis_index("s")
    sg = s // GSZ
    r = s % GSZ
    base = (c * GSZ + sg) * PPG
    ps = base + GSZ * FULL

    pltpu.sync_copy(idx_hbm.at[base + r * FULL], idx_v.at[pl.ds(0, NCHUNK)])
    pltpu.async_copy(table_hbm.at[idx_v.at[0]], buf_a, sem_a)
    pltpu.async_copy(table_hbm.at[idx_v.at[1]], buf_b, sem_b)
    pltpu.async_copy(table_hbm.at[idx_v.at[2]], buf_c, sem_c)
    pltpu.async_copy(table_hbm.at[idx_v.at[3]], buf_d, sem_d)
    pltpu.async_copy(table_hbm.at[idx_v.at[4]], buf_e, sem_e)
    for p in range(1, FULL):
        pltpu.async_copy(
            idx_hbm.at[base + r * FULL + p],
            idx_v.at[pl.ds(p * NCHUNK, NCHUNK)],
            sem_o,
        )
    pltpu.async_copy(
        idx_hbm.at[ps, pl.ds(r * QCH, QCH)],
        idx_v.at[pl.ds(FULL * NCHUNK, QCH)],
        sem_o,
    )
    for p in range(1, FULL):
        pltpu.make_async_copy(
            idx_hbm.at[base + r * FULL + p],
            idx_v.at[pl.ds(p * NCHUNK, NCHUNK)],
            sem_o,
        ).wait()
    pltpu.make_async_copy(
        idx_hbm.at[ps, pl.ds(r * QCH, QCH)],
        idx_v.at[pl.ds(FULL * NCHUNK, QCH)],
        sem_o,
    ).wait()

    def accum(buf, acc):
        def rows(rr, acc):
            r0 = 2 * rr
            acc = tuple(
                acc[h] + buf[r0, pl.ds(LANES * h, LANES)] for h in range(NV)
            )
            return tuple(
                acc[h] + buf[r0 + 1, pl.ds(LANES * h, LANES)]
                for h in range(NV)
            )
        return lax.fori_loop(0, CH // 2, rows, acc)

    zero = jnp.zeros((LANES,), jnp.float32)

    def chunk_step(j, buf, sem, acc):
        pltpu.make_async_copy(table_hbm.at[idx_v.at[j]], buf, sem).wait()
        acc = accum(buf, acc)

        @pl.when(j + NBUF < TCH)
        def _():
            pltpu.async_copy(table_hbm.at[idx_v.at[j + NBUF]], buf, sem)

        done = lax.rem(j + 1, NCHUNK) == 0
        p = lax.div(j + 1, NCHUNK) - 1

        @pl.when(done)
        def _():
            for h in range(NV):
                acc_v[p, pl.ds(LANES * h, LANES)] = acc[h]
            pltpu.async_copy(acc_v.at[p], out_hbm.at[base + r * FULL + p],
                             sem_o)

        return tuple(jnp.where(done, zero, a) for a in acc)

    def ring(i, acc):
        j = NBUF * i
        acc = chunk_step(j, buf_a, sem_a, acc)
        acc = chunk_step(j + 1, buf_b, sem_b, acc)
        acc = chunk_step(j + 2, buf_c, sem_c, acc)
        acc = chunk_step(j + 3, buf_d, sem_d, acc)
        acc = chunk_step(j + 4, buf_e, sem_e, acc)
        return acc

    acc = lax.fori_loop(0, NTRIP, ring, (zero,) * NV)

    for h in range(NV):
        acc_v[FULL, pl.ds(LANES * h, LANES)] = acc[h]
    pltpu.sync_copy(acc_v.at[FULL], part_sh.at[sg, r])

    for p in range(FULL):
        pltpu.make_async_copy(acc_v.at[p], out_hbm.at[base + r * FULL + p],
                              sem_o).wait()
    plsc.subcore_barrier()

    @pl.when(r == 0)
    def _():
        pltpu.sync_copy(part_sh.at[sg], red_v)
        facc = tuple(red_v[0, pl.ds(LANES * h, LANES)] for h in range(NV))
        for q in range(1, GSZ):
            facc = tuple(
                facc[h] + red_v[q, pl.ds(LANES * h, LANES)] for h in range(NV)
            )
        for h in range(NV):
            acc_v[FULL, pl.ds(LANES * h, LANES)] = facc[h]
        pltpu.sync_copy(acc_v.at[FULL], out_hbm.at[ps])


def kernel(input, embedding_table):
    idx = jnp.transpose(input).reshape(HIST, NCHUNK, CH).astype(jnp.int32)
    out = _embed_sum(idx, embedding_table)
    return out.reshape(1, HIST * H)
